# Initial kernel scaffold; baseline (speedup 1.0000x reference)
import jax
import jax.numpy as jnp
from jax.experimental import pallas as pl


def kernel(x, edge_index, edge_attr, W1, b1, W2, b2):
    raise NotImplementedError("write your pallas kernel here")


# speedup vs baseline: 16.8067x; 16.8067x over previous
import functools

import jax
import jax.numpy as jnp
from jax import lax
from jax.experimental import pallas as pl
from jax.experimental.pallas import tpu as pltpu
from jax.experimental.pallas import tpu_sc as plsc

N_NODES = 10000
N_EDGES = 320000
D_FEAT = 128
HIDDEN = 64
N_CLASSES = 40
CP = 48

NC = 2
NS = 16
NW = NC * NS
EPW = N_EDGES // NW
B = 80
K = EPW // B

CHUNK = 624
TAIL = N_NODES - (NS - 1) * CHUNK


def _sc_mesh():
    return plsc.VectorSubcoreMesh(core_axis_name="c", subcore_axis_name="s")


def _deg_body(dst_hbm, ew_hbm, out_hbm, dst_v, ew_v, zbuf, acc):
    cid = lax.axis_index("c")
    sid = lax.axis_index("s")
    wid = sid * NC + cid

    def zb(i, _):
        zbuf[pl.ds(i * 16, 16)] = jnp.zeros((16,), jnp.float32)
        return 0

    lax.fori_loop(0, TAIL // 16, zb, 0)

    @pl.when(sid < NS - 1)
    def _():
        pltpu.sync_copy(zbuf.at[pl.ds(0, CHUNK)], acc.at[pl.ds(sid * CHUNK, CHUNK)])

    @pl.when(sid == NS - 1)
    def _():
        pltpu.sync_copy(zbuf, acc.at[pl.ds((NS - 1) * CHUNK, TAIL)])

    plsc.subcore_barrier()

    pltpu.sync_copy(dst_hbm.at[wid], dst_v)
    pltpu.sync_copy(ew_hbm.at[wid], ew_v)

    def bb(j, _):
        pltpu.sync_copy(ew_v.at[j], acc.at[dst_v.at[j]], add=True)
        return 0

    lax.fori_loop(0, K, bb, 0)
    plsc.subcore_barrier()

    @pl.when(sid < NS - 1)
    def _():
        pltpu.sync_copy(acc.at[pl.ds(sid * CHUNK, CHUNK)], zbuf.at[pl.ds(0, CHUNK)])
        pltpu.sync_copy(zbuf.at[pl.ds(0, CHUNK)],
                        out_hbm.at[pl.ds(cid * N_NODES + sid * CHUNK, CHUNK)])

    @pl.when(sid == NS - 1)
    def _():
        pltpu.sync_copy(acc.at[pl.ds((NS - 1) * CHUNK, TAIL)], zbuf)
        pltpu.sync_copy(zbuf,
                        out_hbm.at[pl.ds(cid * N_NODES + (NS - 1) * CHUNK, TAIL)])


def _sc_degree(dst3, ew3):
    kern = pl.kernel(
        _deg_body,
        out_type=jax.ShapeDtypeStruct((NC * N_NODES,), jnp.float32),
        mesh=_sc_mesh(),
        compiler_params=pltpu.CompilerParams(use_tc_tiling_on_sc=False),
        scratch_types=[
            pltpu.VMEM((K, B), jnp.int32),
            pltpu.VMEM((K, B), jnp.float32),
            pltpu.VMEM((TAIL,), jnp.float32),
            pltpu.VMEM_SHARED((N_NODES,), jnp.float32),
        ],
        name="sc_degree",
    )
    return kern(dst3, ew3)


def _edge_body(D, h_hbm, src_hbm, dst_hbm, ew_hbm, out_hbm,
               src_v, dst_v, ew_v, rows_v, zbuf, acc):
    cid = lax.axis_index("c")
    sid = lax.axis_index("s")
    wid = sid * NC + cid
    nD = D // 16

    def zrow(i, _):
        for c in range(nD):
            zbuf[i, pl.ds(c * 16, 16)] = jnp.zeros((16,), jnp.float32)
        return 0

    lax.fori_loop(0, TAIL, zrow, 0)

    @pl.when(sid < NS - 1)
    def _():
        pltpu.sync_copy(zbuf.at[pl.ds(0, CHUNK)], acc.at[pl.ds(sid * CHUNK, CHUNK)])

    @pl.when(sid == NS - 1)
    def _():
        pltpu.sync_copy(zbuf, acc.at[pl.ds((NS - 1) * CHUNK, TAIL)])

    plsc.subcore_barrier()

    pltpu.sync_copy(src_hbm.at[wid], src_v)
    pltpu.sync_copy(dst_hbm.at[wid], dst_v)
    pltpu.sync_copy(ew_hbm.at[wid], ew_v)

    def bb(j, _):
        pltpu.sync_copy(h_hbm.at[src_v.at[j]], rows_v)

        def rb(g, _):
            ewv = ew_v[pl.ds(j * B + g * 16, 16)]
            for l in range(16):
                wv = jnp.full((16,), ewv[l], jnp.float32)
                i = g * 16 + l
                for c in range(nD):
                    rows_v[i, pl.ds(c * 16, 16)] = rows_v[i, pl.ds(c * 16, 16)] * wv
            return 0

        lax.fori_loop(0, B // 16, rb, 0)
        pltpu.sync_copy(rows_v, acc.at[dst_v.at[j]], add=True)
        return 0

    lax.fori_loop(0, K, bb, 0)
    plsc.subcore_barrier()

    @pl.when(sid < NS - 1)
    def _():
        pltpu.sync_copy(acc.at[pl.ds(sid * CHUNK, CHUNK)], zbuf.at[pl.ds(0, CHUNK)])
        pltpu.sync_copy(zbuf.at[pl.ds(0, CHUNK)],
                        out_hbm.at[cid, pl.ds(sid * CHUNK, CHUNK)])

    @pl.when(sid == NS - 1)
    def _():
        pltpu.sync_copy(acc.at[pl.ds((NS - 1) * CHUNK, TAIL)], zbuf)
        pltpu.sync_copy(zbuf, out_hbm.at[cid, pl.ds((NS - 1) * CHUNK, TAIL)])


def _sc_edge_agg(h, src3, dst3, ew3):
    D = h.shape[1]
    kern = pl.kernel(
        functools.partial(_edge_body, D),
        out_type=jax.ShapeDtypeStruct((NC, N_NODES, D), jnp.float32),
        mesh=_sc_mesh(),
        compiler_params=pltpu.CompilerParams(use_tc_tiling_on_sc=False),
        scratch_types=[
            pltpu.VMEM((K, B), jnp.int32),
            pltpu.VMEM((K, B), jnp.int32),
            pltpu.VMEM((EPW,), jnp.float32),
            pltpu.VMEM((B, D), jnp.float32),
            pltpu.VMEM((TAIL, D), jnp.float32),
            pltpu.VMEM_SHARED((N_NODES, D), jnp.float32),
        ],
        name=f"sc_edge_agg_{D}",
    )
    return kern(h, src3, dst3, ew3.reshape(NW, EPW))


BN = 2000


def _dinv_block(d0_ref, d1_ref):
    deg = 1.0 + d0_ref[...] + d1_ref[...]
    return jnp.where(deg > 0, lax.rsqrt(jnp.maximum(deg, 1e-12)), 0.0)


def _tc1_body(x_ref, w_ref, d0_ref, d1_ref, o_ref):
    dinv = _dinv_block(d0_ref, d1_ref)
    h = jnp.dot(x_ref[...], w_ref[...], preferred_element_type=jnp.float32,
                precision=lax.Precision.HIGHEST)
    o_ref[...] = h * dinv


def _tc1(x, W1, d0, d1):
    grid = (N_NODES // BN,)
    return pl.pallas_call(
        _tc1_body,
        grid=grid,
        in_specs=[
            pl.BlockSpec((BN, D_FEAT), lambda i: (i, 0)),
            pl.BlockSpec((D_FEAT, HIDDEN), lambda i: (0, 0)),
            pl.BlockSpec((BN, 1), lambda i: (i, 0)),
            pl.BlockSpec((BN, 1), lambda i: (i, 0)),
        ],
        out_specs=pl.BlockSpec((BN, HIDDEN), lambda i: (i, 0)),
        out_shape=jax.ShapeDtypeStruct((N_NODES, HIDDEN), jnp.float32),
    )(x, W1, d0, d1)


def _tc2_body(a0_ref, a1_ref, h1d_ref, d0_ref, d1_ref, b1_ref, w2_ref, o_ref):
    dinv = _dinv_block(d0_ref, d1_ref)
    z = (a0_ref[...] + a1_ref[...] + h1d_ref[...]) * dinv + b1_ref[...]
    z = jnp.maximum(z, 0.0)
    h2 = jnp.dot(z, w2_ref[...], preferred_element_type=jnp.float32,
                 precision=lax.Precision.HIGHEST)
    o_ref[...] = h2 * dinv


def _tc2(a0, a1, h1d, d0, d1, b1, W2):
    grid = (N_NODES // BN,)
    return pl.pallas_call(
        _tc2_body,
        grid=grid,
        in_specs=[
            pl.BlockSpec((BN, HIDDEN), lambda i: (i, 0)),
            pl.BlockSpec((BN, HIDDEN), lambda i: (i, 0)),
            pl.BlockSpec((BN, HIDDEN), lambda i: (i, 0)),
            pl.BlockSpec((BN, 1), lambda i: (i, 0)),
            pl.BlockSpec((BN, 1), lambda i: (i, 0)),
            pl.BlockSpec((1, HIDDEN), lambda i: (0, 0)),
            pl.BlockSpec((HIDDEN, CP), lambda i: (0, 0)),
        ],
        out_specs=pl.BlockSpec((BN, CP), lambda i: (i, 0)),
        out_shape=jax.ShapeDtypeStruct((N_NODES, CP), jnp.float32),
    )(a0, a1, h1d, d0, d1, b1, W2)


def _tc3_body(a0_ref, a1_ref, h2d_ref, d0_ref, d1_ref, b2_ref, o_ref):
    dinv = _dinv_block(d0_ref, d1_ref)
    logits = (a0_ref[...] + a1_ref[...] + h2d_ref[...]) * dinv + b2_ref[...]
    li = lax.broadcasted_iota(jnp.int32, logits.shape, 1)
    lm = jnp.where(li < N_CLASSES, logits, jnp.float32(-1e30))
    m = jnp.max(lm, axis=1, keepdims=True)
    e = jnp.exp(lm - m)
    s = jnp.sum(e, axis=1, keepdims=True)
    o_ref[...] = logits - m - jnp.log(s)


def _tc3(a0, a1, h2d, d0, d1, b2):
    grid = (N_NODES // BN,)
    return pl.pallas_call(
        _tc3_body,
        grid=grid,
        in_specs=[
            pl.BlockSpec((BN, CP), lambda i: (i, 0)),
            pl.BlockSpec((BN, CP), lambda i: (i, 0)),
            pl.BlockSpec((BN, CP), lambda i: (i, 0)),
            pl.BlockSpec((BN, 1), lambda i: (i, 0)),
            pl.BlockSpec((BN, 1), lambda i: (i, 0)),
            pl.BlockSpec((1, CP), lambda i: (0, 0)),
        ],
        out_specs=pl.BlockSpec((BN, CP), lambda i: (i, 0)),
        out_shape=jax.ShapeDtypeStruct((N_NODES, CP), jnp.float32),
    )(a0, a1, h2d, d0, d1, b2)


def kernel(x, edge_index, edge_attr, W1, b1, W2, b2):
    src = edge_index[0].astype(jnp.int32)
    dst = edge_index[1].astype(jnp.int32)
    src3 = src.reshape(NW, K, B)
    dst3 = dst.reshape(NW, K, B)
    ew3 = edge_attr.astype(jnp.float32).reshape(NW, K, B)

    degp = _sc_degree(dst3, ew3).reshape(NC, N_NODES)
    d0 = degp[0].reshape(N_NODES, 1)
    d1 = degp[1].reshape(N_NODES, 1)

    W2p = jnp.pad(W2.astype(jnp.float32), ((0, 0), (0, CP - N_CLASSES)))
    b2p = jnp.pad(b2.astype(jnp.float32), (0, CP - N_CLASSES)).reshape(1, CP)

    h1d = _tc1(x, W1, d0, d1)
    a1 = _sc_edge_agg(h1d, src3, dst3, ew3)
    h2d = _tc2(a1[0], a1[1], h1d, d0, d1, b1.reshape(1, HIDDEN), W2p)
    a2 = _sc_edge_agg(h2d, src3, dst3, ew3)
    out = _tc3(a2[0], a2[1], h2d, d0, d1, b2p)
    return out[:, :N_CLASSES]


# speedup vs baseline: 19.9633x; 1.1878x over previous
import functools

import jax
import jax.numpy as jnp
from jax import lax
from jax.experimental import pallas as pl
from jax.experimental.pallas import tpu as pltpu
from jax.experimental.pallas import tpu_sc as plsc

N_NODES = 10000
N_EDGES = 320000
D_FEAT = 128
HIDDEN = 64
N_CLASSES = 40
CP = 48

NC = 2
NS = 16
NW = NC * NS
EPW = N_EDGES // NW
B = 80
K = EPW // B
B2 = 128
KB = 79
EPW2 = KB * B2

CHUNK = 624
TAIL = N_NODES - (NS - 1) * CHUNK


def _sc_mesh():
    return plsc.VectorSubcoreMesh(core_axis_name="c", subcore_axis_name="s")


def _deg_body(dst_hbm, ew_hbm, out_hbm, dst_v, ew_v, zbuf, acc):
    cid = lax.axis_index("c")
    sid = lax.axis_index("s")
    wid = sid * NC + cid

    def zb(i, _):
        zbuf[pl.ds(i * 16, 16)] = jnp.zeros((16,), jnp.float32)
        return 0

    lax.fori_loop(0, TAIL // 16, zb, 0)

    @pl.when(sid < NS - 1)
    def _():
        pltpu.sync_copy(zbuf.at[pl.ds(0, CHUNK)], acc.at[pl.ds(sid * CHUNK, CHUNK)])

    @pl.when(sid == NS - 1)
    def _():
        pltpu.sync_copy(zbuf, acc.at[pl.ds((NS - 1) * CHUNK, TAIL)])

    plsc.subcore_barrier()

    pltpu.sync_copy(dst_hbm.at[wid], dst_v)
    pltpu.sync_copy(ew_hbm.at[wid], ew_v)

    def bb(j, _):
        pltpu.sync_copy(ew_v.at[j], acc.at[dst_v.at[j]], add=True)
        return 0

    lax.fori_loop(0, K, bb, 0)
    plsc.subcore_barrier()

    @pl.when(sid < NS - 1)
    def _():
        pltpu.sync_copy(acc.at[pl.ds(sid * CHUNK, CHUNK)], zbuf.at[pl.ds(0, CHUNK)])
        pltpu.sync_copy(zbuf.at[pl.ds(0, CHUNK)],
                        out_hbm.at[pl.ds(cid * N_NODES + sid * CHUNK, CHUNK)])

    @pl.when(sid == NS - 1)
    def _():
        pltpu.sync_copy(acc.at[pl.ds((NS - 1) * CHUNK, TAIL)], zbuf)
        pltpu.sync_copy(zbuf,
                        out_hbm.at[pl.ds(cid * N_NODES + (NS - 1) * CHUNK, TAIL)])


def _sc_degree(dst3, ew3):
    kern = pl.kernel(
        _deg_body,
        out_type=jax.ShapeDtypeStruct((NC * N_NODES,), jnp.float32),
        mesh=_sc_mesh(),
        compiler_params=pltpu.CompilerParams(use_tc_tiling_on_sc=False),
        scratch_types=[
            pltpu.VMEM((K, B), jnp.int32),
            pltpu.VMEM((K, B), jnp.float32),
            pltpu.VMEM((TAIL,), jnp.float32),
            pltpu.VMEM_SHARED((N_NODES,), jnp.float32),
        ],
        name="sc_degree",
    )
    return kern(dst3, ew3)


def _edge_body(D, h_hbm, src_hbm, dst_hbm, ew_hbm, out_hbm,
               src_v, dst_v, ew_v, rows0, rows1, zbuf, acc, gsem, ssem):
    cid = lax.axis_index("c")
    sid = lax.axis_index("s")
    wid = sid * NC + cid
    nD = D // 16

    def zrow(i, _):
        for c in range(nD):
            zbuf[i, pl.ds(c * 16, 16)] = jnp.zeros((16,), jnp.float32)
        return 0

    lax.fori_loop(0, TAIL, zrow, 0)

    @pl.when(sid < NS - 1)
    def _():
        pltpu.sync_copy(zbuf.at[pl.ds(0, CHUNK)], acc.at[pl.ds(sid * CHUNK, CHUNK)])

    @pl.when(sid == NS - 1)
    def _():
        pltpu.sync_copy(zbuf, acc.at[pl.ds((NS - 1) * CHUNK, TAIL)])

    plsc.subcore_barrier()

    pltpu.sync_copy(src_hbm.at[wid], src_v)
    pltpu.sync_copy(dst_hbm.at[wid], dst_v)
    pltpu.sync_copy(ew_hbm.at[wid], ew_v)

    def start_gather(j, buf):
        pltpu.async_copy(h_hbm.at[src_v.at[j]], buf, gsem)

    def wait_gather(j, buf):
        pltpu.make_async_copy(h_hbm.at[src_v.at[j]], buf, gsem).wait()

    def start_scatter(j, buf):
        pltpu.async_copy(buf, acc.at[dst_v.at[j]], ssem, add=True)

    def wait_scatter(j, buf):
        pltpu.make_async_copy(buf, acc.at[dst_v.at[j]], ssem).wait()

    def scale(j, buf):
        def rb(g, _):
            ewv = ew_v[pl.ds(j * B2 + g * 16, 16)]
            for l in range(16):
                wv = jnp.full((16,), ewv[l], jnp.float32)
                i = g * 16 + l
                for c in range(nD):
                    buf[i, pl.ds(c * 16, 16)] = buf[i, pl.ds(c * 16, 16)] * wv
            return 0

        lax.fori_loop(0, B2 // 16, rb, 0)

    start_gather(0, rows0)

    @pl.loop(0, (KB - 1) // 2)
    def _(p):
        j = 2 * p

        @pl.when(j > 0)
        def _():
            wait_scatter(j - 1, rows1)

        start_gather(j + 1, rows1)
        wait_gather(j, rows0)
        scale(j, rows0)
        start_scatter(j, rows0)

        wait_scatter(j, rows0)
        start_gather(j + 2, rows0)
        wait_gather(j + 1, rows1)
        scale(j + 1, rows1)
        start_scatter(j + 1, rows1)

    wait_gather(KB - 1, rows0)
    scale(KB - 1, rows0)
    start_scatter(KB - 1, rows0)
    wait_scatter(KB - 2, rows1)
    wait_scatter(KB - 1, rows0)
    plsc.subcore_barrier()

    @pl.when(sid < NS - 1)
    def _():
        pltpu.sync_copy(acc.at[pl.ds(sid * CHUNK, CHUNK)], zbuf.at[pl.ds(0, CHUNK)])
        pltpu.sync_copy(zbuf.at[pl.ds(0, CHUNK)],
                        out_hbm.at[cid, pl.ds(sid * CHUNK, CHUNK)])

    @pl.when(sid == NS - 1)
    def _():
        pltpu.sync_copy(acc.at[pl.ds((NS - 1) * CHUNK, TAIL)], zbuf)
        pltpu.sync_copy(zbuf, out_hbm.at[cid, pl.ds((NS - 1) * CHUNK, TAIL)])


def _sc_edge_agg(h, src3, dst3, ew3):
    D = h.shape[1]
    kern = pl.kernel(
        functools.partial(_edge_body, D),
        out_type=jax.ShapeDtypeStruct((NC, N_NODES, D), jnp.float32),
        mesh=_sc_mesh(),
        compiler_params=pltpu.CompilerParams(use_tc_tiling_on_sc=False),
        scratch_types=[
            pltpu.VMEM((KB, B2), jnp.int32),
            pltpu.VMEM((KB, B2), jnp.int32),
            pltpu.VMEM((EPW2,), jnp.float32),
            pltpu.VMEM((B2, D), jnp.float32),
            pltpu.VMEM((B2, D), jnp.float32),
            pltpu.VMEM((TAIL, D), jnp.float32),
            pltpu.VMEM_SHARED((N_NODES, D), jnp.float32),
            pltpu.SemaphoreType.DMA,
            pltpu.SemaphoreType.DMA,
        ],
        name=f"sc_edge_agg_{D}",
    )
    return kern(h, src3, dst3, ew3)


BN = 2000


def _dinv_block(d0_ref, d1_ref):
    deg = 1.0 + d0_ref[...] + d1_ref[...]
    return jnp.where(deg > 0, lax.rsqrt(jnp.maximum(deg, 1e-12)), 0.0)


def _tc1_body(x_ref, w_ref, d0_ref, d1_ref, o_ref):
    dinv = _dinv_block(d0_ref, d1_ref)
    h = jnp.dot(x_ref[...], w_ref[...], preferred_element_type=jnp.float32,
                precision=lax.Precision.HIGHEST)
    o_ref[...] = h * dinv


def _tc1(x, W1, d0, d1):
    grid = (N_NODES // BN,)
    return pl.pallas_call(
        _tc1_body,
        grid=grid,
        in_specs=[
            pl.BlockSpec((BN, D_FEAT), lambda i: (i, 0)),
            pl.BlockSpec((D_FEAT, HIDDEN), lambda i: (0, 0)),
            pl.BlockSpec((BN, 1), lambda i: (i, 0)),
            pl.BlockSpec((BN, 1), lambda i: (i, 0)),
        ],
        out_specs=pl.BlockSpec((BN, HIDDEN), lambda i: (i, 0)),
        out_shape=jax.ShapeDtypeStruct((N_NODES, HIDDEN), jnp.float32),
    )(x, W1, d0, d1)


def _tc2_body(a0_ref, a1_ref, h1d_ref, d0_ref, d1_ref, b1_ref, w2_ref, o_ref):
    dinv = _dinv_block(d0_ref, d1_ref)
    z = (a0_ref[...] + a1_ref[...] + h1d_ref[...]) * dinv + b1_ref[...]
    z = jnp.maximum(z, 0.0)
    h2 = jnp.dot(z, w2_ref[...], preferred_element_type=jnp.float32,
                 precision=lax.Precision.HIGHEST)
    o_ref[...] = h2 * dinv


def _tc2(a0, a1, h1d, d0, d1, b1, W2):
    grid = (N_NODES // BN,)
    return pl.pallas_call(
        _tc2_body,
        grid=grid,
        in_specs=[
            pl.BlockSpec((BN, HIDDEN), lambda i: (i, 0)),
            pl.BlockSpec((BN, HIDDEN), lambda i: (i, 0)),
            pl.BlockSpec((BN, HIDDEN), lambda i: (i, 0)),
            pl.BlockSpec((BN, 1), lambda i: (i, 0)),
            pl.BlockSpec((BN, 1), lambda i: (i, 0)),
            pl.BlockSpec((1, HIDDEN), lambda i: (0, 0)),
            pl.BlockSpec((HIDDEN, CP), lambda i: (0, 0)),
        ],
        out_specs=pl.BlockSpec((BN, CP), lambda i: (i, 0)),
        out_shape=jax.ShapeDtypeStruct((N_NODES, CP), jnp.float32),
    )(a0, a1, h1d, d0, d1, b1, W2)


def _tc3_body(a0_ref, a1_ref, h2d_ref, d0_ref, d1_ref, b2_ref, o_ref):
    dinv = _dinv_block(d0_ref, d1_ref)
    logits = (a0_ref[...] + a1_ref[...] + h2d_ref[...]) * dinv + b2_ref[...]
    li = lax.broadcasted_iota(jnp.int32, logits.shape, 1)
    lm = jnp.where(li < N_CLASSES, logits, jnp.float32(-1e30))
    m = jnp.max(lm, axis=1, keepdims=True)
    e = jnp.exp(lm - m)
    s = jnp.sum(e, axis=1, keepdims=True)
    o_ref[...] = logits - m - jnp.log(s)


def _tc3(a0, a1, h2d, d0, d1, b2):
    grid = (N_NODES // BN,)
    return pl.pallas_call(
        _tc3_body,
        grid=grid,
        in_specs=[
            pl.BlockSpec((BN, CP), lambda i: (i, 0)),
            pl.BlockSpec((BN, CP), lambda i: (i, 0)),
            pl.BlockSpec((BN, CP), lambda i: (i, 0)),
            pl.BlockSpec((BN, 1), lambda i: (i, 0)),
            pl.BlockSpec((BN, 1), lambda i: (i, 0)),
            pl.BlockSpec((1, CP), lambda i: (0, 0)),
        ],
        out_specs=pl.BlockSpec((BN, CP), lambda i: (i, 0)),
        out_shape=jax.ShapeDtypeStruct((N_NODES, CP), jnp.float32),
    )(a0, a1, h2d, d0, d1, b2)


def kernel(x, edge_index, edge_attr, W1, b1, W2, b2):
    src = edge_index[0].astype(jnp.int32)
    dst = edge_index[1].astype(jnp.int32)
    ew = edge_attr.astype(jnp.float32)

    pad = NW * EPW2 - N_EDGES
    srcp = jnp.concatenate([src, jnp.zeros((pad,), jnp.int32)]).reshape(NW, KB, B2)
    dstp = jnp.concatenate([dst, jnp.zeros((pad,), jnp.int32)]).reshape(NW, KB, B2)
    ewp = jnp.concatenate([ew, jnp.zeros((pad,), jnp.float32)]).reshape(NW, EPW2)

    dst3 = dst.reshape(NW, K, B)
    ew3 = ew.reshape(NW, K, B)

    degp = _sc_degree(dst3, ew3).reshape(NC, N_NODES)
    d0 = degp[0].reshape(N_NODES, 1)
    d1 = degp[1].reshape(N_NODES, 1)

    W2p = jnp.pad(W2.astype(jnp.float32), ((0, 0), (0, CP - N_CLASSES)))
    b2p = jnp.pad(b2.astype(jnp.float32), (0, CP - N_CLASSES)).reshape(1, CP)

    h1d = _tc1(x, W1, d0, d1)
    a1 = _sc_edge_agg(h1d, srcp, dstp, ewp)
    h2d = _tc2(a1[0], a1[1], h1d, d0, d1, b1.reshape(1, HIDDEN), W2p)
    a2 = _sc_edge_agg(h2d, srcp, dstp, ewp)
    out = _tc3(a2[0], a2[1], h2d, d0, d1, b2p)
    return out[:, :N_CLASSES]


# speedup vs baseline: 21.2103x; 1.0625x over previous
import functools

import jax
import jax.numpy as jnp
from jax import lax
from jax.experimental import pallas as pl
from jax.experimental.pallas import tpu as pltpu
from jax.experimental.pallas import tpu_sc as plsc

N_NODES = 10000
N_EDGES = 320000
D_FEAT = 128
HIDDEN = 64
N_CLASSES = 40
CP = 48

NC = 2
NS = 16
NW = NC * NS
EPW = N_EDGES // NW
B = 80
K = EPW // B
B2 = 128
KB = 79
EPW2 = KB * B2

CHUNK = 624
TAIL = N_NODES - (NS - 1) * CHUNK
ZR = 208


def _sc_mesh():
    return plsc.VectorSubcoreMesh(core_axis_name="c", subcore_axis_name="s")


def _deg_body(dst_hbm, ew_hbm, out_hbm, dst_v, ew_v, zbuf, acc):
    cid = lax.axis_index("c")
    sid = lax.axis_index("s")
    wid = sid * NC + cid

    def zb(i, _):
        zbuf[pl.ds(i * 16, 16)] = jnp.zeros((16,), jnp.float32)
        return 0

    lax.fori_loop(0, TAIL // 16, zb, 0)

    @pl.when(sid < NS - 1)
    def _():
        pltpu.sync_copy(zbuf.at[pl.ds(0, CHUNK)], acc.at[pl.ds(sid * CHUNK, CHUNK)])

    @pl.when(sid == NS - 1)
    def _():
        pltpu.sync_copy(zbuf, acc.at[pl.ds((NS - 1) * CHUNK, TAIL)])

    plsc.subcore_barrier()

    pltpu.sync_copy(dst_hbm.at[wid], dst_v)
    pltpu.sync_copy(ew_hbm.at[wid], ew_v)

    def bb(j, _):
        pltpu.sync_copy(ew_v.at[j], acc.at[dst_v.at[j]], add=True)
        return 0

    lax.fori_loop(0, K, bb, 0)
    plsc.subcore_barrier()

    @pl.when(sid < NS - 1)
    def _():
        pltpu.sync_copy(acc.at[pl.ds(sid * CHUNK, CHUNK)], zbuf.at[pl.ds(0, CHUNK)])
        pltpu.sync_copy(zbuf.at[pl.ds(0, CHUNK)],
                        out_hbm.at[pl.ds(cid * N_NODES + sid * CHUNK, CHUNK)])

    @pl.when(sid == NS - 1)
    def _():
        pltpu.sync_copy(acc.at[pl.ds((NS - 1) * CHUNK, TAIL)], zbuf)
        pltpu.sync_copy(zbuf,
                        out_hbm.at[pl.ds(cid * N_NODES + (NS - 1) * CHUNK, TAIL)])


def _sc_degree(dst3, ew3):
    kern = pl.kernel(
        _deg_body,
        out_type=jax.ShapeDtypeStruct((NC * N_NODES,), jnp.float32),
        mesh=_sc_mesh(),
        compiler_params=pltpu.CompilerParams(use_tc_tiling_on_sc=False),
        scratch_types=[
            pltpu.VMEM((K, B), jnp.int32),
            pltpu.VMEM((K, B), jnp.float32),
            pltpu.VMEM((TAIL,), jnp.float32),
            pltpu.VMEM_SHARED((N_NODES,), jnp.float32),
        ],
        name="sc_degree",
    )
    return kern(dst3, ew3)


def _edge_body(D, h_hbm, src_hbm, dst_hbm, ew_hbm, out_hbm,
               src_v, dst_v, ew_v, rows0, rows1, rows2, zbuf, acc, gsem, ssem):
    cid = lax.axis_index("c")
    sid = lax.axis_index("s")
    wid = sid * NC + cid
    nD = D // 16

    def zrow(i, _):
        for c in range(nD):
            zbuf[i, pl.ds(c * 16, 16)] = jnp.zeros((16,), jnp.float32)
        return 0

    lax.fori_loop(0, ZR, zrow, 0)

    for r in range(3):
        pltpu.sync_copy(zbuf, acc.at[pl.ds(sid * CHUNK + r * ZR, ZR)])

    @pl.when(sid == NS - 1)
    def _():
        pltpu.sync_copy(zbuf.at[pl.ds(0, TAIL - 3 * ZR)],
                        acc.at[pl.ds(NS * CHUNK, TAIL - 3 * ZR)])

    plsc.subcore_barrier()

    pltpu.sync_copy(src_hbm.at[wid], src_v)
    pltpu.sync_copy(dst_hbm.at[wid], dst_v)
    pltpu.sync_copy(ew_hbm.at[wid], ew_v)

    def start_gather(j, buf):
        pltpu.async_copy(h_hbm.at[src_v.at[j]], buf, gsem)

    def wait_gather(j, buf):
        pltpu.make_async_copy(h_hbm.at[src_v.at[j]], buf, gsem).wait()

    def start_scatter(j, buf):
        pltpu.async_copy(buf, acc.at[dst_v.at[j]], ssem, add=True)

    def wait_scatter(j, buf):
        pltpu.make_async_copy(buf, acc.at[dst_v.at[j]], ssem).wait()

    def scale(j, buf):
        def rb(g, _):
            ewv = ew_v[pl.ds(j * B2 + g * 16, 16)]
            for l in range(16):
                wv = jnp.full((16,), ewv[l], jnp.float32)
                i = g * 16 + l
                for c in range(nD):
                    buf[i, pl.ds(c * 16, 16)] = buf[i, pl.ds(c * 16, 16)] * wv
            return 0

        lax.fori_loop(0, B2 // 16, rb, 0)

    bufs = (rows0, rows1, rows2)
    start_gather(0, rows0)
    start_gather(1, rows1)

    def step(j):
        bj = bufs[j % 3]
        wait_gather(j, bj)
        scale(j, bj)
        start_scatter(j, bj)
        if j >= 1:
            wait_scatter(j - 1, bufs[(j - 1) % 3])
        if j + 2 < KB:
            start_gather(j + 2, bufs[(j + 2) % 3])

    @pl.loop(0, KB // 3)
    def _(p):
        j0 = 3 * p

        def stepd(j, bj, bprev, do_wait, do_gather):
            wait_gather(j, bj)
            scale(j, bj)
            start_scatter(j, bj)

            @pl.when(do_wait)
            def _():
                wait_scatter(j - 1, bprev)

            @pl.when(do_gather)
            def _():
                start_gather(j + 2, bprev)

        stepd(j0, bufs[0], bufs[2], j0 >= 1, j0 + 2 < KB)
        stepd(j0 + 1, bufs[1], bufs[0], j0 + 1 >= 1, j0 + 3 < KB)
        stepd(j0 + 2, bufs[2], bufs[1], j0 + 2 >= 1, j0 + 4 < KB)

    jt = KB - 1
    wait_gather(jt, bufs[jt % 3])
    scale(jt, bufs[jt % 3])
    start_scatter(jt, bufs[jt % 3])
    wait_scatter(jt - 1, bufs[(jt - 1) % 3])
    wait_scatter(jt, bufs[jt % 3])
    plsc.subcore_barrier()

    for r in range(3):
        pltpu.sync_copy(acc.at[pl.ds(sid * CHUNK + r * ZR, ZR)], zbuf)
        pltpu.sync_copy(zbuf, out_hbm.at[cid, pl.ds(sid * CHUNK + r * ZR, ZR)])

    @pl.when(sid == NS - 1)
    def _():
        pltpu.sync_copy(acc.at[pl.ds(NS * CHUNK, TAIL - 3 * ZR)],
                        zbuf.at[pl.ds(0, TAIL - 3 * ZR)])
        pltpu.sync_copy(zbuf.at[pl.ds(0, TAIL - 3 * ZR)],
                        out_hbm.at[cid, pl.ds(NS * CHUNK, TAIL - 3 * ZR)])


def _sc_edge_agg(h, src3, dst3, ew3):
    D = h.shape[1]
    kern = pl.kernel(
        functools.partial(_edge_body, D),
        out_type=jax.ShapeDtypeStruct((NC, N_NODES, D), jnp.float32),
        mesh=_sc_mesh(),
        compiler_params=pltpu.CompilerParams(use_tc_tiling_on_sc=False),
        scratch_types=[
            pltpu.VMEM((KB, B2), jnp.int32),
            pltpu.VMEM((KB, B2), jnp.int32),
            pltpu.VMEM((EPW2,), jnp.float32),
            pltpu.VMEM((B2, D), jnp.float32),
            pltpu.VMEM((B2, D), jnp.float32),
            pltpu.VMEM((B2, D), jnp.float32),
            pltpu.VMEM((ZR, D), jnp.float32),
            pltpu.VMEM_SHARED((N_NODES, D), jnp.float32),
            pltpu.SemaphoreType.DMA,
            pltpu.SemaphoreType.DMA,
        ],
        name=f"sc_edge_agg_{D}",
    )
    return kern(h, src3, dst3, ew3)


BN = 2000


def _dinv_block(d0_ref, d1_ref):
    deg = 1.0 + d0_ref[...] + d1_ref[...]
    return jnp.where(deg > 0, lax.rsqrt(jnp.maximum(deg, 1e-12)), 0.0)


def _tc1_body(x_ref, w_ref, d0_ref, d1_ref, o_ref):
    dinv = _dinv_block(d0_ref, d1_ref)
    h = jnp.dot(x_ref[...], w_ref[...], preferred_element_type=jnp.float32,
                precision=lax.Precision.HIGHEST)
    o_ref[...] = h * dinv


def _tc1(x, W1, d0, d1):
    grid = (N_NODES // BN,)
    return pl.pallas_call(
        _tc1_body,
        grid=grid,
        in_specs=[
            pl.BlockSpec((BN, D_FEAT), lambda i: (i, 0)),
            pl.BlockSpec((D_FEAT, HIDDEN), lambda i: (0, 0)),
            pl.BlockSpec((BN, 1), lambda i: (i, 0)),
            pl.BlockSpec((BN, 1), lambda i: (i, 0)),
        ],
        out_specs=pl.BlockSpec((BN, HIDDEN), lambda i: (i, 0)),
        out_shape=jax.ShapeDtypeStruct((N_NODES, HIDDEN), jnp.float32),
    )(x, W1, d0, d1)


def _tc2_body(a0_ref, a1_ref, h1d_ref, d0_ref, d1_ref, b1_ref, w2_ref, o_ref):
    dinv = _dinv_block(d0_ref, d1_ref)
    z = (a0_ref[...] + a1_ref[...] + h1d_ref[...]) * dinv + b1_ref[...]
    z = jnp.maximum(z, 0.0)
    h2 = jnp.dot(z, w2_ref[...], preferred_element_type=jnp.float32,
                 precision=lax.Precision.HIGHEST)
    o_ref[...] = h2 * dinv


def _tc2(a0, a1, h1d, d0, d1, b1, W2):
    grid = (N_NODES // BN,)
    return pl.pallas_call(
        _tc2_body,
        grid=grid,
        in_specs=[
            pl.BlockSpec((BN, HIDDEN), lambda i: (i, 0)),
            pl.BlockSpec((BN, HIDDEN), lambda i: (i, 0)),
            pl.BlockSpec((BN, HIDDEN), lambda i: (i, 0)),
            pl.BlockSpec((BN, 1), lambda i: (i, 0)),
            pl.BlockSpec((BN, 1), lambda i: (i, 0)),
            pl.BlockSpec((1, HIDDEN), lambda i: (0, 0)),
            pl.BlockSpec((HIDDEN, CP), lambda i: (0, 0)),
        ],
        out_specs=pl.BlockSpec((BN, CP), lambda i: (i, 0)),
        out_shape=jax.ShapeDtypeStruct((N_NODES, CP), jnp.float32),
    )(a0, a1, h1d, d0, d1, b1, W2)


def _tc3_body(a0_ref, a1_ref, h2d_ref, d0_ref, d1_ref, b2_ref, o_ref):
    dinv = _dinv_block(d0_ref, d1_ref)
    logits = (a0_ref[...] + a1_ref[...] + h2d_ref[...]) * dinv + b2_ref[...]
    li = lax.broadcasted_iota(jnp.int32, logits.shape, 1)
    lm = jnp.where(li < N_CLASSES, logits, jnp.float32(-1e30))
    m = jnp.max(lm, axis=1, keepdims=True)
    e = jnp.exp(lm - m)
    s = jnp.sum(e, axis=1, keepdims=True)
    o_ref[...] = logits - m - jnp.log(s)


def _tc3(a0, a1, h2d, d0, d1, b2):
    grid = (N_NODES // BN,)
    return pl.pallas_call(
        _tc3_body,
        grid=grid,
        in_specs=[
            pl.BlockSpec((BN, CP), lambda i: (i, 0)),
            pl.BlockSpec((BN, CP), lambda i: (i, 0)),
            pl.BlockSpec((BN, CP), lambda i: (i, 0)),
            pl.BlockSpec((BN, 1), lambda i: (i, 0)),
            pl.BlockSpec((BN, 1), lambda i: (i, 0)),
            pl.BlockSpec((1, CP), lambda i: (0, 0)),
        ],
        out_specs=pl.BlockSpec((BN, CP), lambda i: (i, 0)),
        out_shape=jax.ShapeDtypeStruct((N_NODES, CP), jnp.float32),
    )(a0, a1, h2d, d0, d1, b2)


def kernel(x, edge_index, edge_attr, W1, b1, W2, b2):
    src = edge_index[0].astype(jnp.int32)
    dst = edge_index[1].astype(jnp.int32)
    ew = edge_attr.astype(jnp.float32)

    pad = NW * EPW2 - N_EDGES
    srcp = jnp.concatenate([src, jnp.zeros((pad,), jnp.int32)]).reshape(NW, KB, B2)
    dstp = jnp.concatenate([dst, jnp.zeros((pad,), jnp.int32)]).reshape(NW, KB, B2)
    ewp = jnp.concatenate([ew, jnp.zeros((pad,), jnp.float32)]).reshape(NW, EPW2)

    dst3 = dst.reshape(NW, K, B)
    ew3 = ew.reshape(NW, K, B)

    degp = _sc_degree(dst3, ew3).reshape(NC, N_NODES)
    d0 = degp[0].reshape(N_NODES, 1)
    d1 = degp[1].reshape(N_NODES, 1)

    W2p = jnp.pad(W2.astype(jnp.float32), ((0, 0), (0, CP - N_CLASSES)))
    b2p = jnp.pad(b2.astype(jnp.float32), (0, CP - N_CLASSES)).reshape(1, CP)

    h1d = _tc1(x, W1, d0, d1)
    a1 = _sc_edge_agg(h1d, srcp, dstp, ewp)
    h2d = _tc2(a1[0], a1[1], h1d, d0, d1, b1.reshape(1, HIDDEN), W2p)
    a2 = _sc_edge_agg(h2d, srcp, dstp, ewp)
    out = _tc3(a2[0], a2[1], h2d, d0, d1, b2p)
    return out[:, :N_CLASSES]


# speedup vs baseline: 25.7673x; 1.2149x over previous
import functools

import jax
import jax.numpy as jnp
from jax import lax
from jax.experimental import pallas as pl
from jax.experimental.pallas import tpu as pltpu
from jax.experimental.pallas import tpu_sc as plsc

N_NODES = 10000
N_EDGES = 320000
D_FEAT = 128
HIDDEN = 64
N_CLASSES = 40
CP = 48

NC = 2
NS = 16
NW = NC * NS
EPW = N_EDGES // NW
B = 80
K = EPW // B
B2 = 96
KB = 105
EPW2 = KB * B2

CHUNK = 624
TAIL = N_NODES - (NS - 1) * CHUNK
ZR = 208


def _sc_mesh():
    return plsc.VectorSubcoreMesh(core_axis_name="c", subcore_axis_name="s")


def _deg_body(dst_hbm, ew_hbm, out_hbm, dst_v, ew_v, zbuf, acc):
    cid = lax.axis_index("c")
    sid = lax.axis_index("s")
    wid = sid * NC + cid

    def zb(i, _):
        zbuf[pl.ds(i * 16, 16)] = jnp.zeros((16,), jnp.float32)
        return 0

    lax.fori_loop(0, TAIL // 16, zb, 0)

    @pl.when(sid < NS - 1)
    def _():
        pltpu.sync_copy(zbuf.at[pl.ds(0, CHUNK)], acc.at[pl.ds(sid * CHUNK, CHUNK)])

    @pl.when(sid == NS - 1)
    def _():
        pltpu.sync_copy(zbuf, acc.at[pl.ds((NS - 1) * CHUNK, TAIL)])

    plsc.subcore_barrier()

    pltpu.sync_copy(dst_hbm.at[wid], dst_v)
    pltpu.sync_copy(ew_hbm.at[wid], ew_v)

    def bb(j, _):
        pltpu.sync_copy(ew_v.at[j], acc.at[dst_v.at[j]], add=True)
        return 0

    lax.fori_loop(0, K, bb, 0)
    plsc.subcore_barrier()

    @pl.when(sid < NS - 1)
    def _():
        pltpu.sync_copy(acc.at[pl.ds(sid * CHUNK, CHUNK)], zbuf.at[pl.ds(0, CHUNK)])
        pltpu.sync_copy(zbuf.at[pl.ds(0, CHUNK)],
                        out_hbm.at[pl.ds(cid * N_NODES + sid * CHUNK, CHUNK)])

    @pl.when(sid == NS - 1)
    def _():
        pltpu.sync_copy(acc.at[pl.ds((NS - 1) * CHUNK, TAIL)], zbuf)
        pltpu.sync_copy(zbuf,
                        out_hbm.at[pl.ds(cid * N_NODES + (NS - 1) * CHUNK, TAIL)])


def _sc_degree(dst3, ew3):
    kern = pl.kernel(
        _deg_body,
        out_type=jax.ShapeDtypeStruct((NC * N_NODES,), jnp.float32),
        mesh=_sc_mesh(),
        compiler_params=pltpu.CompilerParams(use_tc_tiling_on_sc=False),
        scratch_types=[
            pltpu.VMEM((K, B), jnp.int32),
            pltpu.VMEM((K, B), jnp.float32),
            pltpu.VMEM((TAIL,), jnp.float32),
            pltpu.VMEM_SHARED((N_NODES,), jnp.float32),
        ],
        name="sc_degree",
    )
    return kern(dst3, ew3)


def _copy_tile_range(sid, copy_chunk, copy_rem):
    for r in range(6):
        copy_chunk(r * B2)

    @pl.when(sid < NS - 1)
    def _():
        copy_rem(6 * B2, 48)

    @pl.when(sid == NS - 1)
    def _():
        copy_rem(6 * B2, 64)


def _edge_body(D, h_hbm, src_hbm, dst_hbm, ew_hbm, out_hbm,
               src_v, dst_v, ew_v, rows0, rows1, rows2, table, acc, gsem, ssem):
    cid = lax.axis_index("c")
    sid = lax.axis_index("s")
    wid = sid * NC + cid
    nD = D // 16
    tb = sid * CHUNK

    def zrow(i, _):
        for c in range(nD):
            rows0[i, pl.ds(c * 16, 16)] = jnp.zeros((16,), jnp.float32)
        return 0

    lax.fori_loop(0, B2, zrow, 0)

    def zero_chunk(off):
        pltpu.sync_copy(rows0, acc.at[pl.ds(tb + off, B2)])

    def zero_rem(off, n):
        pltpu.sync_copy(rows0.at[pl.ds(0, n)], acc.at[pl.ds(tb + off, n)])

    _copy_tile_range(sid, zero_chunk, zero_rem)

    def load_chunk(off):
        pltpu.sync_copy(h_hbm.at[pl.ds(tb + off, B2)], rows1)
        pltpu.sync_copy(rows1, table.at[pl.ds(tb + off, B2)])

    def load_rem(off, n):
        pltpu.sync_copy(h_hbm.at[pl.ds(tb + off, n)], rows1.at[pl.ds(0, n)])
        pltpu.sync_copy(rows1.at[pl.ds(0, n)], table.at[pl.ds(tb + off, n)])

    _copy_tile_range(sid, load_chunk, load_rem)

    plsc.subcore_barrier()

    pltpu.sync_copy(src_hbm.at[wid], src_v)
    pltpu.sync_copy(dst_hbm.at[wid], dst_v)
    pltpu.sync_copy(ew_hbm.at[wid], ew_v)

    def start_gather(j, buf):
        pltpu.async_copy(table.at[src_v.at[j]], buf, gsem)

    def wait_gather(j, buf):
        pltpu.make_async_copy(table.at[src_v.at[j]], buf, gsem).wait()

    def start_scatter(j, buf):
        pltpu.async_copy(buf, acc.at[dst_v.at[j]], ssem, add=True)

    def wait_scatter(j, buf):
        pltpu.make_async_copy(buf, acc.at[dst_v.at[j]], ssem).wait()

    def scale(j, buf):
        def rb(g, _):
            ewv = ew_v[pl.ds(j * B2 + g * 16, 16)]
            for l in range(16):
                wv = jnp.full((16,), ewv[l], jnp.float32)
                i = g * 16 + l
                for c in range(nD):
                    buf[i, pl.ds(c * 16, 16)] = buf[i, pl.ds(c * 16, 16)] * wv
            return 0

        lax.fori_loop(0, B2 // 16, rb, 0)

    bufs = (rows0, rows1, rows2)
    start_gather(0, rows0)
    start_gather(1, rows1)

    @pl.loop(0, KB // 3)
    def _(p):
        j0 = 3 * p

        def stepd(j, bj, bprev, do_wait, do_gather):
            wait_gather(j, bj)
            scale(j, bj)
            start_scatter(j, bj)

            @pl.when(do_wait)
            def _():
                wait_scatter(j - 1, bprev)

            @pl.when(do_gather)
            def _():
                start_gather(j + 2, bprev)

        stepd(j0, bufs[0], bufs[2], j0 >= 1, j0 + 2 < KB)
        stepd(j0 + 1, bufs[1], bufs[0], True, j0 + 3 < KB)
        stepd(j0 + 2, bufs[2], bufs[1], True, j0 + 4 < KB)

    wait_scatter(KB - 1, bufs[(KB - 1) % 3])
    plsc.subcore_barrier()

    def out_chunk(off):
        pltpu.sync_copy(acc.at[pl.ds(tb + off, B2)], rows0)
        pltpu.sync_copy(rows0, out_hbm.at[cid, pl.ds(tb + off, B2)])

    def out_rem(off, n):
        pltpu.sync_copy(acc.at[pl.ds(tb + off, n)], rows0.at[pl.ds(0, n)])
        pltpu.sync_copy(rows0.at[pl.ds(0, n)], out_hbm.at[cid, pl.ds(tb + off, n)])

    _copy_tile_range(sid, out_chunk, out_rem)


def _sc_edge_agg(h, src3, dst3, ew3):
    D = h.shape[1]
    kern = pl.kernel(
        functools.partial(_edge_body, D),
        out_type=jax.ShapeDtypeStruct((NC, N_NODES, D), jnp.float32),
        mesh=_sc_mesh(),
        compiler_params=pltpu.CompilerParams(use_tc_tiling_on_sc=False),
        scratch_types=[
            pltpu.VMEM((KB, B2), jnp.int32),
            pltpu.VMEM((KB, B2), jnp.int32),
            pltpu.VMEM((EPW2,), jnp.float32),
            pltpu.VMEM((B2, D), jnp.float32),
            pltpu.VMEM((B2, D), jnp.float32),
            pltpu.VMEM((B2, D), jnp.float32),
            pltpu.VMEM_SHARED((N_NODES, D), jnp.float32),
            pltpu.VMEM_SHARED((N_NODES, D), jnp.float32),
            pltpu.SemaphoreType.DMA,
            pltpu.SemaphoreType.DMA,
        ],
        name=f"sc_edge_agg_{D}",
    )
    return kern(h, src3, dst3, ew3)


BN = 2000


def _dinv_block(d0_ref, d1_ref):
    deg = 1.0 + d0_ref[...] + d1_ref[...]
    return jnp.where(deg > 0, lax.rsqrt(jnp.maximum(deg, 1e-12)), 0.0)


def _tc1_body(x_ref, w_ref, d0_ref, d1_ref, o_ref):
    dinv = _dinv_block(d0_ref, d1_ref)
    h = jnp.dot(x_ref[...], w_ref[...], preferred_element_type=jnp.float32,
                precision=lax.Precision.HIGHEST)
    o_ref[...] = h * dinv


def _tc1(x, W1, d0, d1):
    grid = (N_NODES // BN,)
    return pl.pallas_call(
        _tc1_body,
        grid=grid,
        in_specs=[
            pl.BlockSpec((BN, D_FEAT), lambda i: (i, 0)),
            pl.BlockSpec((D_FEAT, HIDDEN), lambda i: (0, 0)),
            pl.BlockSpec((BN, 1), lambda i: (i, 0)),
            pl.BlockSpec((BN, 1), lambda i: (i, 0)),
        ],
        out_specs=pl.BlockSpec((BN, HIDDEN), lambda i: (i, 0)),
        out_shape=jax.ShapeDtypeStruct((N_NODES, HIDDEN), jnp.float32),
    )(x, W1, d0, d1)


def _tc2_body(a0_ref, a1_ref, h1d_ref, d0_ref, d1_ref, b1_ref, w2_ref, o_ref):
    dinv = _dinv_block(d0_ref, d1_ref)
    z = (a0_ref[...] + a1_ref[...] + h1d_ref[...]) * dinv + b1_ref[...]
    z = jnp.maximum(z, 0.0)
    h2 = jnp.dot(z, w2_ref[...], preferred_element_type=jnp.float32,
                 precision=lax.Precision.HIGHEST)
    o_ref[...] = h2 * dinv


def _tc2(a0, a1, h1d, d0, d1, b1, W2):
    grid = (N_NODES // BN,)
    return pl.pallas_call(
        _tc2_body,
        grid=grid,
        in_specs=[
            pl.BlockSpec((BN, HIDDEN), lambda i: (i, 0)),
            pl.BlockSpec((BN, HIDDEN), lambda i: (i, 0)),
            pl.BlockSpec((BN, HIDDEN), lambda i: (i, 0)),
            pl.BlockSpec((BN, 1), lambda i: (i, 0)),
            pl.BlockSpec((BN, 1), lambda i: (i, 0)),
            pl.BlockSpec((1, HIDDEN), lambda i: (0, 0)),
            pl.BlockSpec((HIDDEN, CP), lambda i: (0, 0)),
        ],
        out_specs=pl.BlockSpec((BN, CP), lambda i: (i, 0)),
        out_shape=jax.ShapeDtypeStruct((N_NODES, CP), jnp.float32),
    )(a0, a1, h1d, d0, d1, b1, W2)


def _tc3_body(a0_ref, a1_ref, h2d_ref, d0_ref, d1_ref, b2_ref, o_ref):
    dinv = _dinv_block(d0_ref, d1_ref)
    logits = (a0_ref[...] + a1_ref[...] + h2d_ref[...]) * dinv + b2_ref[...]
    li = lax.broadcasted_iota(jnp.int32, logits.shape, 1)
    lm = jnp.where(li < N_CLASSES, logits, jnp.float32(-1e30))
    m = jnp.max(lm, axis=1, keepdims=True)
    e = jnp.exp(lm - m)
    s = jnp.sum(e, axis=1, keepdims=True)
    o_ref[...] = logits - m - jnp.log(s)


def _tc3(a0, a1, h2d, d0, d1, b2):
    grid = (N_NODES // BN,)
    return pl.pallas_call(
        _tc3_body,
        grid=grid,
        in_specs=[
            pl.BlockSpec((BN, CP), lambda i: (i, 0)),
            pl.BlockSpec((BN, CP), lambda i: (i, 0)),
            pl.BlockSpec((BN, CP), lambda i: (i, 0)),
            pl.BlockSpec((BN, 1), lambda i: (i, 0)),
            pl.BlockSpec((BN, 1), lambda i: (i, 0)),
            pl.BlockSpec((1, CP), lambda i: (0, 0)),
        ],
        out_specs=pl.BlockSpec((BN, CP), lambda i: (i, 0)),
        out_shape=jax.ShapeDtypeStruct((N_NODES, CP), jnp.float32),
    )(a0, a1, h2d, d0, d1, b2)


def kernel(x, edge_index, edge_attr, W1, b1, W2, b2):
    src = edge_index[0].astype(jnp.int32)
    dst = edge_index[1].astype(jnp.int32)
    ew = edge_attr.astype(jnp.float32)

    pad = NW * EPW2 - N_EDGES
    srcp = jnp.concatenate([src, jnp.zeros((pad,), jnp.int32)]).reshape(NW, KB, B2)
    dstp = jnp.concatenate([dst, jnp.zeros((pad,), jnp.int32)]).reshape(NW, KB, B2)
    ewp = jnp.concatenate([ew, jnp.zeros((pad,), jnp.float32)]).reshape(NW, EPW2)

    dst3 = dst.reshape(NW, K, B)
    ew3 = ew.reshape(NW, K, B)

    degp = _sc_degree(dst3, ew3).reshape(NC, N_NODES)
    d0 = degp[0].reshape(N_NODES, 1)
    d1 = degp[1].reshape(N_NODES, 1)

    W2p = jnp.pad(W2.astype(jnp.float32), ((0, 0), (0, CP - N_CLASSES)))
    b2p = jnp.pad(b2.astype(jnp.float32), (0, CP - N_CLASSES)).reshape(1, CP)

    h1d = _tc1(x, W1, d0, d1)
    a1 = _sc_edge_agg(h1d, srcp, dstp, ewp)
    h2d = _tc2(a1[0], a1[1], h1d, d0, d1, b1.reshape(1, HIDDEN), W2p)
    a2 = _sc_edge_agg(h2d, srcp, dstp, ewp)
    out = _tc3(a2[0], a2[1], h2d, d0, d1, b2p)
    return out[:, :N_CLASSES]
